# Initial kernel scaffold; baseline (speedup 1.0000x reference)
import jax
import jax.numpy as jnp
from jax.experimental import pallas as pl


def kernel(x, edge_index):
    raise NotImplementedError("write your pallas kernel here")


# speedup vs baseline: 6.2323x; 6.2323x over previous
import functools

import jax
import jax.numpy as jnp
from jax import lax
from jax.experimental import pallas as pl
from jax.experimental.pallas import tpu as pltpu
from jax.experimental.pallas import tpu_sc as plsc

N = 10000
E = 320000
D = 128
NC = 2
NS = 16
NW = NC * NS
EPW = E // NW
C = 80
NCHUNK = EPW // C
XROWS = N // NW
LANES = 16
VPR = D // LANES


@functools.partial(
    pl.kernel,
    out_type=jax.ShapeDtypeStruct((N + E, D), jnp.float32),
    mesh=plsc.VectorSubcoreMesh(core_axis_name="c", subcore_axis_name="s"),
    scratch_types=[
        pltpu.VMEM((C,), jnp.int32),
        pltpu.VMEM((C,), jnp.int32),
        pltpu.VMEM((C, D), jnp.float32),
        pltpu.VMEM((C, D), jnp.float32),
        pltpu.SemaphoreType.DMA,
    ],
)
def _gunpool(x_hbm, src_hbm, dst_hbm, out_hbm, src_v, dst_v, buf_a, buf_b, sem):
    cid = lax.axis_index("c")
    sid = lax.axis_index("s")
    wid = sid * NC + cid
    ebase = wid * EPW

    @pl.when(wid == 0)
    def _copy_x():
        pltpu.sync_copy(x_hbm, out_hbm.at[pl.ds(0, N)])

    def chunk_body(i, carry):
        off = ebase + i * C
        pltpu.sync_copy(src_hbm.at[pl.ds(off, C)], src_v)
        pltpu.sync_copy(dst_hbm.at[pl.ds(off, C)], dst_v)
        cp_a = pltpu.async_copy(x_hbm.at[src_v], buf_a, sem)
        cp_b = pltpu.async_copy(x_hbm.at[dst_v], buf_b, sem)
        cp_a.wait()
        cp_b.wait()

        def row_body(r, rc):
            for j in range(VPR):
                a = buf_a[r, pl.ds(j * LANES, LANES)]
                b = buf_b[r, pl.ds(j * LANES, LANES)]
                buf_b[r, pl.ds(j * LANES, LANES)] = 0.5 * (a + b)
            return rc

        lax.fori_loop(0, C, row_body, 0)
        pltpu.sync_copy(buf_b, out_hbm.at[pl.ds(N + off, C)])
        return carry

    lax.fori_loop(0, NCHUNK, chunk_body, 0)


def kernel(x, edge_index):
    out = _gunpool(x[0], edge_index[0], edge_index[1])
    return out[None]


# speedup vs baseline: 10.3690x; 1.6637x over previous
import functools

import jax
import jax.numpy as jnp
from jax import lax
from jax.experimental import pallas as pl
from jax.experimental.pallas import tpu as pltpu
from jax.experimental.pallas import tpu_sc as plsc

N = 10000
E = 320000
D = 128
NC = 2
NS = 16
NW = NC * NS
EPW = E // NW
C = 40
NCHUNK = EPW // C
NT = NCHUNK // 2
LANES = 16
VPR = D // LANES


def _avg(a_ref, b_ref, o_ref):
    def row_body(r, rc):
        for j in range(VPR):
            s = pl.ds(j * LANES, LANES)
            o_ref[r, s] = 0.5 * (a_ref[r, s] + b_ref[r, s])
        return rc

    lax.fori_loop(0, C, row_body, 0)


@functools.partial(
    pl.kernel,
    out_type=jax.ShapeDtypeStruct((N + E, D), jnp.float32),
    mesh=plsc.VectorSubcoreMesh(core_axis_name="c", subcore_axis_name="s"),
    scratch_types=[
        pltpu.VMEM((EPW,), jnp.int32),
        pltpu.VMEM((EPW,), jnp.int32),
        pltpu.VMEM((C, D), jnp.float32),
        pltpu.VMEM((C, D), jnp.float32),
        pltpu.VMEM((C, D), jnp.float32),
        pltpu.VMEM((C, D), jnp.float32),
        pltpu.VMEM((C, D), jnp.float32),
        pltpu.VMEM((C, D), jnp.float32),
        pltpu.SemaphoreType.DMA,
        pltpu.SemaphoreType.DMA,
        pltpu.SemaphoreType.DMA,
        pltpu.SemaphoreType.DMA,
    ],
)
def _gunpool(x_hbm, src_hbm, dst_hbm, out_hbm,
             src_all, dst_all, a0, b0, o0, a1, b1, o1,
             sem_g0, sem_g1, sem_s0, sem_s1):
    cid = lax.axis_index("c")
    sid = lax.axis_index("s")
    wid = sid * NC + cid
    ebase = wid * EPW
    obase = N + ebase

    @pl.when(wid == 0)
    def _copy_x():
        pltpu.sync_copy(x_hbm, out_hbm.at[pl.ds(0, N)])

    pltpu.sync_copy(src_hbm.at[pl.ds(ebase, EPW)], src_all)
    pltpu.sync_copy(dst_hbm.at[pl.ds(ebase, EPW)], dst_all)

    def fire_gather(off, a_buf, b_buf, sem):
        pltpu.async_copy(x_hbm.at[src_all.at[pl.ds(off, C)]], a_buf, sem)
        pltpu.async_copy(x_hbm.at[dst_all.at[pl.ds(off, C)]], b_buf, sem)

    def wait_gather(off, a_buf, b_buf, sem):
        pltpu.make_async_copy(x_hbm.at[src_all.at[pl.ds(off, C)]], a_buf, sem).wait()
        pltpu.make_async_copy(x_hbm.at[dst_all.at[pl.ds(off, C)]], b_buf, sem).wait()

    fire_gather(0, a0, b0, sem_g0)

    def body(t, carry):
        off0 = (2 * t) * C
        off1 = off0 + C
        off2 = off1 + C

        fire_gather(off1, a1, b1, sem_g1)

        wait_gather(off0, a0, b0, sem_g0)

        @pl.when(t > 0)
        def _drain_s0():
            pltpu.make_async_copy(o0, out_hbm.at[pl.ds(obase, C)], sem_s0).wait()

        _avg(a0, b0, o0)
        pltpu.async_copy(o0, out_hbm.at[pl.ds(obase + off0, C)], sem_s0)

        @pl.when(t < NT - 1)
        def _prefetch_next():
            fire_gather(off2, a0, b0, sem_g0)

        wait_gather(off1, a1, b1, sem_g1)

        @pl.when(t > 0)
        def _drain_s1():
            pltpu.make_async_copy(o1, out_hbm.at[pl.ds(obase, C)], sem_s1).wait()

        _avg(a1, b1, o1)
        pltpu.async_copy(o1, out_hbm.at[pl.ds(obase + off1, C)], sem_s1)
        return carry

    lax.fori_loop(0, NT, body, 0)

    pltpu.make_async_copy(o0, out_hbm.at[pl.ds(obase, C)], sem_s0).wait()
    pltpu.make_async_copy(o1, out_hbm.at[pl.ds(obase, C)], sem_s1).wait()


def kernel(x, edge_index):
    out = _gunpool(x[0], edge_index[0], edge_index[1])
    return out[None]
